# baseline (speedup 1.0000x reference)
import functools

import jax
import jax.numpy as jnp
from jax import lax
from jax.experimental import pallas as pl
from jax.experimental.pallas import tpu as pltpu
from jax.experimental.pallas import tpu_sc as plsc

N_FIELDS = 26
VOCAB = 100000
EMBED_DIM = 16
BATCH = 16384

NC = 2
NS = 16
NW = NC * NS

GWIN = 128
VB = 4096
VOCAB_PAD = 25 * VB
NF_A, NF_B = 16, 10

BLKL = 256


def _tr_body(t_ref, o_ref):
    t = t_ref[...].T
    o_ref[0] = pltpu.bitcast(t.astype(jnp.bfloat16), jnp.float32)


def _tc_transpose(tables_2d, fgp, row0):
    return pl.pallas_call(
        _tr_body,
        grid=(fgp, VOCAB_PAD // VB),
        compiler_params=pltpu.CompilerParams(
            dimension_semantics=("parallel", "parallel")),
        in_specs=[pl.BlockSpec((128, VB), lambda g, j: (g + row0, j))],
        out_specs=pl.BlockSpec((1, VB // 2, 128), lambda g, j: (g, j, 0)),
        out_shape=jax.ShapeDtypeStruct((fgp, VOCAB_PAD // 2, 128),
                                       jnp.float32),
    )(tables_2d)


def _sc_gather(flat_tables, gidx, total_rows, group):
    rows_per_w = total_rows // NW
    nwin = rows_per_w // GWIN
    ngroups = nwin // group
    gchunk = group * GWIN
    assert nwin % group == 0 and ngroups % 2 == 0
    mesh = plsc.VectorSubcoreMesh(core_axis_name="c", subcore_axis_name="s")

    @functools.partial(
        pl.kernel,
        mesh=mesh,
        compiler_params=pltpu.CompilerParams(use_tc_tiling_on_sc=False),
        out_type=jax.ShapeDtypeStruct((total_rows, EMBED_DIM), jnp.float32),
        scratch_types=[
            pltpu.VMEM((nwin, GWIN), jnp.int32),
            pltpu.VMEM((gchunk, EMBED_DIM), jnp.float32),
            pltpu.VMEM((gchunk, EMBED_DIM), jnp.float32),
            pltpu.SemaphoreType.DMA,
            pltpu.SemaphoreType.DMA,
            pltpu.SemaphoreType.DMA,
            pltpu.SemaphoreType.DMA,
        ],
    )
    def k(tbl_hbm, idx_hbm, out_hbm, idx_v, rows_a, rows_b, gsem_a, gsem_b,
          wsem_a, wsem_b):
        wid = lax.axis_index("s") * NC + lax.axis_index("c")
        pltpu.sync_copy(idx_hbm.at[wid], idx_v)
        bufs = ((rows_a, gsem_a, wsem_a), (rows_b, gsem_b, wsem_b))

        @pl.loop(0, ngroups, step=2)
        def _(go):
            for s in range(2):
                buf, gsem, wsem = bufs[s]
                g = go + s
                row0 = wid * rows_per_w + g * gchunk

                @pl.when(go > 0)
                def _():
                    pltpu.make_async_copy(
                        buf, out_hbm.at[pl.ds(0, gchunk)], wsem).wait()

                cps = [
                    pltpu.make_async_copy(
                        tbl_hbm.at[idx_v.at[g * group + t]],
                        buf.at[pl.ds(t * GWIN, GWIN)],
                        gsem)
                    for t in range(group)
                ]
                for cp in cps:
                    cp.start()
                for cp in cps:
                    cp.wait()
                pltpu.make_async_copy(
                    buf, out_hbm.at[pl.ds(row0, gchunk)], wsem).start()

        for buf, _, wsem in bufs:
            pltpu.make_async_copy(
                buf, out_hbm.at[pl.ds(0, gchunk)], wsem).wait()

    return k(flat_tables, gidx)


def _group_mat(transpose=False):
    li = lax.broadcasted_iota(jnp.int32, (128, 8), 0) // EMBED_DIM
    ci = lax.broadcasted_iota(jnp.int32, (128, 8), 1)
    g = (li == ci).astype(jnp.float32)
    return g.T if transpose else g


def _red_body(sta_ref, stb_ref, par_ref, w_ref, b_ref, o_ref):
    ew = jnp.concatenate([sta_ref[...], stb_ref[...]], axis=0)
    ui = lax.bitcast_convert_type(ew, jnp.int32)
    lo = lax.bitcast_convert_type(ui << 16, jnp.float32)
    hi = lax.bitcast_convert_type(
        ui & jnp.int32(-65536), jnp.float32)
    par8 = par_ref[...].reshape(N_FIELDS * BLKL, 8).astype(jnp.float32)
    par = lax.dot_general(par8, _group_mat(True), (((1,), (0,)), ((), ())),
                          preferred_element_type=jnp.float32)
    par = par.reshape(N_FIELDS, BLKL, 128)
    e = jnp.where(par > 0.5, hi, lo)
    s = jnp.sum(e, axis=0)
    q = jnp.sum(e * e, axis=0)
    t = jnp.sum(e * w_ref[...], axis=0)
    z = t + 0.5 * (s * s - q)
    y8 = lax.dot_general(z, _group_mat(), (((1,), (0,)), ((), ())),
                         preferred_element_type=jnp.float32)
    o_ref[...] = jax.nn.sigmoid(y8 + b_ref[0])


def _tc_reduce(stga, stgb, parr, w128, b):
    nlines = BATCH // 8
    return pl.pallas_call(
        _red_body,
        grid=(nlines // BLKL,),
        compiler_params=pltpu.CompilerParams(
            dimension_semantics=("parallel",)),
        in_specs=[
            pl.BlockSpec((NF_A, BLKL, 128), lambda i: (0, i, 0)),
            pl.BlockSpec((NF_B, BLKL, 128), lambda i: (0, i, 0)),
            pl.BlockSpec((N_FIELDS, BLKL, 8), lambda i: (0, i, 0)),
            pl.BlockSpec((N_FIELDS, 1, 128), lambda i: (0, 0, 0)),
            pl.BlockSpec(memory_space=pltpu.SMEM),
        ],
        out_specs=pl.BlockSpec((BLKL, 8), lambda i: (i, 0)),
        out_shape=jax.ShapeDtypeStruct((nlines, 8), jnp.float32),
    )(stga, stgb, parr, w128, b)


def kernel(x, tables, W, b):
    tables_t = jnp.swapaxes(tables, 1, 2)
    tables_2d = tables_t.reshape(N_FIELDS * EMBED_DIM, VOCAB)
    tbl_a = _tc_transpose(tables_2d, 2, 0)
    tbl_b = _tc_transpose(tables_2d, 2, 2)
    flat_a = tbl_a.reshape(VOCAB_PAD * 8, EMBED_DIM)
    flat_b = tbl_b.reshape(VOCAB_PAD * 8, EMBED_DIM)
    xt = x.T.astype(jnp.int32)
    f = jnp.arange(N_FIELDS, dtype=jnp.int32)[:, None]
    local = jnp.where(f < NF_A, f, f - NF_A)
    offs = (local // 8) * (VOCAB_PAD // 2 * 8) + (local % 8)
    gidx = (xt >> 1) * 8 + offs
    gidx_a = gidx[:NF_A].reshape(NW, -1, GWIN)
    gidx_b = gidx[NF_A:].reshape(NW, -1, GWIN)
    parr = (xt & 1).reshape(N_FIELDS, BATCH // 8, 8)
    staged_a = _sc_gather(flat_a, gidx_a, NF_A * BATCH, 8)
    staged_b = _sc_gather(flat_b, gidx_b, NF_B * BATCH, 10)
    stg_a = staged_a.reshape(NF_A, BATCH // 8, 128)
    stg_b = staged_b.reshape(NF_B, BATCH // 8, 128)
    w128 = jnp.tile(W.reshape(N_FIELDS, 1, EMBED_DIM), (1, 1, 8))
    out8 = _tc_reduce(stg_a, stg_b, parr, w128, b)
    return out8.reshape(BATCH)


# speedup vs baseline: 1.3216x; 1.3216x over previous
import functools

import jax
import jax.numpy as jnp
from jax import lax
from jax.experimental import pallas as pl
from jax.experimental.pallas import tpu as pltpu
from jax.experimental.pallas import tpu_sc as plsc

N_FIELDS = 26
VOCAB = 100000
EMBED_DIM = 16
BATCH = 16384

NC = 2
NS = 16
NW = NC * NS

GWIN = 128
VB = 4096
VOCAB_PAD = 25 * VB
NF_A, NF_B = 16, 10
NE_A, NE_B = 8, 5

BLKL = 256


def _tr_body(t_ref, o_ref):
    e = t_ref[...]
    pk = pltpu.bitcast(e.astype(jnp.bfloat16), jnp.float32)
    o_ref[0] = pk.T


def _tc_transpose(tables_2d, row0):
    return pl.pallas_call(
        _tr_body,
        grid=(1, VOCAB_PAD // VB),
        compiler_params=pltpu.CompilerParams(
            dimension_semantics=("parallel", "parallel")),
        in_specs=[pl.BlockSpec((256, VB), lambda g, j: (g + row0 // 2, j))],
        out_specs=pl.BlockSpec((1, VB, 128), lambda g, j: (g, j, 0)),
        out_shape=jax.ShapeDtypeStruct((1, VOCAB_PAD, 128), jnp.float32),
    )(tables_2d)


def _sc_gather(flat_tables, gidx, total_rows, group):
    rows_per_w = total_rows // NW
    nwin = rows_per_w // GWIN
    ngroups = nwin // group
    gchunk = group * GWIN
    assert nwin % group == 0 and ngroups % 2 == 0
    mesh = plsc.VectorSubcoreMesh(core_axis_name="c", subcore_axis_name="s")

    @functools.partial(
        pl.kernel,
        mesh=mesh,
        compiler_params=pltpu.CompilerParams(use_tc_tiling_on_sc=False),
        out_type=jax.ShapeDtypeStruct((total_rows, EMBED_DIM), jnp.float32),
        scratch_types=[
            pltpu.VMEM((nwin, GWIN), jnp.int32),
            pltpu.VMEM((gchunk, EMBED_DIM), jnp.float32),
            pltpu.VMEM((gchunk, EMBED_DIM), jnp.float32),
            pltpu.SemaphoreType.DMA,
            pltpu.SemaphoreType.DMA,
            pltpu.SemaphoreType.DMA,
            pltpu.SemaphoreType.DMA,
        ],
    )
    def k(tbl_hbm, idx_hbm, out_hbm, idx_v, rows_a, rows_b, gsem_a, gsem_b,
          wsem_a, wsem_b):
        wid = lax.axis_index("s") * NC + lax.axis_index("c")
        pltpu.sync_copy(idx_hbm.at[wid], idx_v)
        bufs = ((rows_a, gsem_a, wsem_a), (rows_b, gsem_b, wsem_b))

        @pl.loop(0, ngroups, step=2)
        def _(go):
            for s in range(2):
                buf, gsem, wsem = bufs[s]
                g = go + s
                row0 = wid * rows_per_w + g * gchunk

                @pl.when(go > 0)
                def _():
                    pltpu.make_async_copy(
                        buf, out_hbm.at[pl.ds(0, gchunk)], wsem).wait()

                cps = [
                    pltpu.make_async_copy(
                        tbl_hbm.at[idx_v.at[g * group + t]],
                        buf.at[pl.ds(t * GWIN, GWIN)],
                        gsem)
                    for t in range(group)
                ]
                for cp in cps:
                    cp.start()
                for cp in cps:
                    cp.wait()
                pltpu.make_async_copy(
                    buf, out_hbm.at[pl.ds(row0, gchunk)], wsem).start()

        for buf, _, wsem in bufs:
            pltpu.make_async_copy(
                buf, out_hbm.at[pl.ds(0, gchunk)], wsem).wait()

    return k(flat_tables, gidx)


def _group_mat():
    li = lax.broadcasted_iota(jnp.int32, (128, 8), 0) // EMBED_DIM
    ci = lax.broadcasted_iota(jnp.int32, (128, 8), 1)
    return (li == ci).astype(jnp.float32)


def _unpack(ew, ne):
    ev = ew[:ne]
    od = pltpu.roll(ew[ne:], 120, 2)
    allw = jnp.concatenate([ev, od], axis=0)
    ui = lax.bitcast_convert_type(allw, jnp.int32)
    lo = lax.bitcast_convert_type(ui << 16, jnp.float32)
    hi = lax.bitcast_convert_type(ui & jnp.int32(-65536), jnp.float32)
    return lo, hi


def _red_body(sta_ref, stb_ref, wlo_ref, whi_ref, b_ref, o_ref):
    lo_a, hi_a = _unpack(sta_ref[...], NE_A)
    lo_b, hi_b = _unpack(stb_ref[...], NE_B)
    lo = jnp.concatenate([lo_a, lo_b], axis=0)
    hi = jnp.concatenate([hi_a, hi_b], axis=0)
    sl = jnp.sum(lo, axis=0)
    sh = jnp.sum(hi, axis=0)
    q = jnp.sum(lo * lo + hi * hi, axis=0)
    tt = jnp.sum(lo * wlo_ref[...] + hi * whi_ref[...], axis=0)
    z = tt + 0.5 * (sl * sl + sh * sh - q)
    lanes = lax.broadcasted_iota(jnp.int32, (BLKL, 128), 1)
    zm = jnp.where(lanes % EMBED_DIM < 8, z, 0.0)
    y8 = lax.dot_general(zm, _group_mat(), (((1,), (0,)), ((), ())),
                         preferred_element_type=jnp.float32)
    o_ref[...] = jax.nn.sigmoid(y8 + b_ref[0])


def _tc_reduce(stga, stgb, wlo, whi, b):
    nlines = BATCH // 8
    return pl.pallas_call(
        _red_body,
        grid=(nlines // BLKL,),
        compiler_params=pltpu.CompilerParams(
            dimension_semantics=("parallel",)),
        in_specs=[
            pl.BlockSpec((NF_A, BLKL, 128), lambda i: (0, i, 0)),
            pl.BlockSpec((NF_B, BLKL, 128), lambda i: (0, i, 0)),
            pl.BlockSpec((N_FIELDS, 1, 128), lambda i: (0, 0, 0)),
            pl.BlockSpec((N_FIELDS, 1, 128), lambda i: (0, 0, 0)),
            pl.BlockSpec(memory_space=pltpu.SMEM),
        ],
        out_specs=pl.BlockSpec((BLKL, 8), lambda i: (i, 0)),
        out_shape=jax.ShapeDtypeStruct((nlines, 8), jnp.float32),
    )(stga, stgb, wlo, whi, b)


def _field_order():
    orda = [f for f in range(NF_A) if f % 2 == 0] + \
           [f for f in range(NF_A) if f % 2 == 1]
    ordb = [f for f in range(NF_A, N_FIELDS) if f % 2 == 0] + \
           [f for f in range(NF_A, N_FIELDS) if f % 2 == 1]
    return orda + ordb


def kernel(x, tables, W, b):
    tables_t = jnp.swapaxes(tables, 1, 2)
    tables_2d = tables_t.reshape(N_FIELDS * EMBED_DIM, VOCAB)
    tbl_a = _tc_transpose(tables_2d, 0)
    tbl_b = _tc_transpose(tables_2d, 2)
    flat_a = tbl_a.reshape(VOCAB_PAD * 8, EMBED_DIM)
    flat_b = tbl_b.reshape(VOCAB_PAD * 8, EMBED_DIM)

    order = jnp.array(_field_order(), dtype=jnp.int32)
    xt = x.T.astype(jnp.int32)
    f = jnp.arange(N_FIELDS, dtype=jnp.int32)[:, None]
    slot_pair = (f % NF_A) // 2
    gidx = (xt * 8 + slot_pair)[order]
    gidx_a = gidx[:NF_A].reshape(NW, -1, GWIN)
    gidx_b = gidx[NF_A:].reshape(NW, -1, GWIN)

    staged_a = _sc_gather(flat_a, gidx_a, NF_A * BATCH, 16)
    staged_b = _sc_gather(flat_b, gidx_b, NF_B * BATCH, 20)
    stg_a = staged_a.reshape(NF_A, BATCH // 8, 128)
    stg_b = staged_b.reshape(NF_B, BATCH // 8, 128)

    w2 = W.reshape(N_FIELDS, EMBED_DIM)[order]
    lane_w = jnp.arange(128) % EMBED_DIM
    valid = lane_w < 8
    wlo = jnp.where(valid[None, None, :],
                    w2[:, jnp.minimum(2 * lane_w, 15)][:, None, :], 0.0)
    whi = jnp.where(valid[None, None, :],
                    w2[:, jnp.minimum(2 * lane_w + 1, 15)][:, None, :], 0.0)

    out8 = _tc_reduce(stg_a, stg_b, wlo, whi, b)
    return out8.reshape(BATCH)
